# baseline (speedup 1.0000x reference)
import functools

import jax
import jax.numpy as jnp
from jax import lax
from jax.experimental import pallas as pl
from jax.experimental.pallas import tpu as pltpu
from jax.experimental.pallas import tpu_sc as plsc

HIDDEN = 768
REL = 53
NUM_BAGS = 16
TOTAL = 32768
BAG = TOTAL // NUM_BAGS
HALF_BAGS = NUM_BAGS // 2
HALF = TOTAL // 2


def _sweep_kernel(x_ref, k_ref, rel_ref, bias_ref, p_ref, idx_ref):
    b = pl.program_id(0)
    logits = jnp.dot(x_ref[...], rel_ref[...],
                     preferred_element_type=jnp.float32) + bias_ref[...]
    m = jnp.max(logits, axis=1, keepdims=True)
    e = jnp.exp(logits - m)
    p = e / jnp.sum(e, axis=1, keepdims=True)
    p_ref[...] = p
    score = jnp.sum(p * k_ref[...], axis=1, keepdims=True)

    lm = jnp.max(score)
    ids = lax.broadcasted_iota(jnp.int32, (BAG, 1), 0)
    lj = jnp.min(jnp.where(score == lm, ids, BAG))
    idx_ref[b] = b * BAG + lj


def _sweep_half(x, knowledge, rel_mat, bias2d, half):
    return pl.pallas_call(
        _sweep_kernel,
        grid=(HALF_BAGS,),
        in_specs=[
            pl.BlockSpec((BAG, HIDDEN), lambda i: (i + half * HALF_BAGS, 0)),
            pl.BlockSpec((BAG, REL), lambda i: (i + half * HALF_BAGS, 0)),
            pl.BlockSpec((HIDDEN, REL), lambda i: (0, 0)),
            pl.BlockSpec((1, REL), lambda i: (0, 0)),
        ],
        out_specs=[
            pl.BlockSpec((BAG, REL), lambda i: (i, 0)),
            pl.BlockSpec(memory_space=pltpu.MemorySpace.SMEM),
        ],
        out_shape=[
            jax.ShapeDtypeStruct((HALF, REL), jnp.float32),
            jax.ShapeDtypeStruct((HALF_BAGS,), jnp.int32),
        ],
    )(x, knowledge, rel_mat, bias2d)


def _sc_gather(idx, probs):
    mesh = plsc.VectorSubcoreMesh(core_axis_name="c", subcore_axis_name="s")

    @functools.partial(
        pl.kernel,
        mesh=mesh,
        out_type=jax.ShapeDtypeStruct((HALF_BAGS, REL), jnp.float32),
        scratch_types=[
            pltpu.VMEM((HALF_BAGS,), jnp.int32),
            pltpu.VMEM((REL,), jnp.float32),
        ],
    )
    def gather(idx_hbm, probs_hbm, out_hbm, idx_v, row_v):
        wid = lax.axis_index("s") * 2 + lax.axis_index("c")

        @pl.when(wid < HALF_BAGS)
        def _():
            pltpu.sync_copy(idx_hbm, idx_v)
            iv = idx_v[...]
            j = iv[0]
            for l in range(1, HALF_BAGS):
                j = jnp.where(wid == l, iv[l], j)
            pltpu.sync_copy(probs_hbm.at[j], row_v)
            pltpu.sync_copy(row_v, out_hbm.at[wid])

    return gather(idx, probs)


@jax.jit
def _selector(x, knowledge, rel_mat, bias2d):
    probs_a, idx_a = _sweep_half(x, knowledge, rel_mat, bias2d, 0)
    rows_a = _sc_gather(idx_a, probs_a)
    probs_b, idx_b = _sweep_half(x, knowledge, rel_mat, bias2d, 1)
    rows_b = _sc_gather(idx_b, probs_b)
    return jnp.concatenate([rows_a, rows_b], axis=0)


def kernel(x, scope, knowledge, rel_mat, bias):
    del scope
    out = _selector(x, knowledge, rel_mat, bias.reshape(1, REL))
    return out, rel_mat


# speedup vs baseline: 1.0484x; 1.0484x over previous
import functools

import jax
import jax.numpy as jnp
from jax import lax
from jax.experimental import pallas as pl
from jax.experimental.pallas import tpu as pltpu
from jax.experimental.pallas import tpu_sc as plsc

HIDDEN = 768
REL = 53
NUM_BAGS = 16
TOTAL = 32768
BAG = TOTAL // NUM_BAGS


def _sweep_kernel(x_ref, k_ref, rel_ref, bias_ref, p_ref, idx_ref):
    b = pl.program_id(0)
    logits = jnp.dot(x_ref[...], rel_ref[...],
                     preferred_element_type=jnp.float32) + bias_ref[...]
    m = jnp.max(logits, axis=1, keepdims=True)
    e = jnp.exp(logits - m)
    p = e / jnp.sum(e, axis=1, keepdims=True)
    p_ref[...] = p
    score = jnp.sum(p * k_ref[...], axis=1, keepdims=True)

    lm = jnp.max(score)
    ids = lax.broadcasted_iota(jnp.int32, (BAG, 1), 0)
    lj = jnp.min(jnp.where(score == lm, ids, BAG))
    idx_ref[b] = b * BAG + lj


def _sc_gather(idx, probs):
    mesh = plsc.VectorSubcoreMesh(core_axis_name="c", subcore_axis_name="s")

    @functools.partial(
        pl.kernel,
        mesh=mesh,
        out_type=jax.ShapeDtypeStruct((NUM_BAGS, REL), jnp.float32),
        scratch_types=[
            pltpu.VMEM((NUM_BAGS,), jnp.int32),
            pltpu.VMEM((REL,), jnp.float32),
        ],
    )
    def gather(idx_hbm, probs_hbm, out_hbm, idx_v, row_v):
        wid = lax.axis_index("s") * 2 + lax.axis_index("c")

        @pl.when(wid < NUM_BAGS)
        def _():
            pltpu.sync_copy(idx_hbm, idx_v)
            iv = idx_v[...]
            j = iv[0]
            for l in range(1, NUM_BAGS):
                j = jnp.where(wid == l, iv[l], j)
            pltpu.sync_copy(probs_hbm.at[j], row_v)
            pltpu.sync_copy(row_v, out_hbm.at[wid])

    return gather(idx, probs)


@jax.jit
def _selector(x, knowledge, rel_mat, bias2d):
    probs, idx = pl.pallas_call(
        _sweep_kernel,
        grid=(NUM_BAGS,),
        in_specs=[
            pl.BlockSpec((BAG, HIDDEN), lambda i: (i, 0)),
            pl.BlockSpec((BAG, REL), lambda i: (i, 0)),
            pl.BlockSpec((HIDDEN, REL), lambda i: (0, 0)),
            pl.BlockSpec((1, REL), lambda i: (0, 0)),
        ],
        out_specs=[
            pl.BlockSpec((BAG, REL), lambda i: (i, 0)),
            pl.BlockSpec(memory_space=pltpu.MemorySpace.SMEM),
        ],
        out_shape=[
            jax.ShapeDtypeStruct((TOTAL, REL), jnp.float32),
            jax.ShapeDtypeStruct((NUM_BAGS,), jnp.int32),
        ],
    )(x, knowledge, rel_mat, bias2d)

    return _sc_gather(idx, probs)


def kernel(x, scope, knowledge, rel_mat, bias):
    del scope
    out = _selector(x, knowledge, rel_mat, bias.reshape(1, REL))
    return out, rel_mat
